# baseline (speedup 1.0000x reference)
import functools

import jax
import jax.numpy as jnp
from jax import lax
from jax.experimental import pallas as pl
from jax.experimental.pallas import tpu as pltpu
from jax.experimental.pallas import tpu_sc as plsc

N = 16384
INPUT_DIM = 40960
INTER_DIM = 641
OUT_DIM = 32
NNZ = 524288

NC = 2
NS = 16
LANES = 16
NW = NC * NS
CHUNK = NNZ // NW
BLK = 128
NBLK = CHUNK // BLK
GROUPS = BLK // LANES
CROWS = CHUNK // 128
PACK = 128 // OUT_DIM
NPACK = N // PACK
ROWS_PER_TEC = NPACK // NS
NBUF = 2


def _sc_body(pa_hbm, val_hbm, w_hbm, z_hbm, out_hbm,
             acc, w_v, pa_v, val_v, rows3, bidx3, colprev, sems, zsems):
    cid = lax.axis_index("c")
    sid = lax.axis_index("s")
    wid = sid * NC + cid

    base = wid * CROWS
    pltpu.sync_copy(pa_hbm.at[pl.ds(base, CROWS)], pa_v)
    pltpu.sync_copy(val_hbm.at[pl.ds(base, CROWS)], val_v)
    pltpu.sync_copy(w_hbm, w_v)
    pltpu.sync_copy(z_hbm.at[pl.ds(sid * ROWS_PER_TEC, ROWS_PER_TEC)],
                    acc.at[pl.ds(sid * ROWS_PER_TEC, ROWS_PER_TEC)])
    for u0 in range(NBUF):
        pltpu.make_async_copy(z_hbm.at[pl.ds(0, BLK)], rows3.at[u0],
                              zsems.at[u0]).start()
    zero16i = jnp.zeros((LANES,), jnp.int32)
    zero16 = jnp.zeros((LANES,), jnp.float32)
    for u0 in range(NBUF):
        pltpu.make_async_copy(z_hbm.at[pl.ds(0, BLK)], rows3.at[u0],
                              zsems.at[u0]).wait()
        for g in range(GROUPS):
            colprev[u0, pl.ds(g * LANES, LANES)] = zero16i
    plsc.subcore_barrier()

    def build(j, u):
        crow = j
        cbase = 0
        for g in range(GROUPS):
            sl = pl.ds(cbase + g * LANES, LANES)
            gsl = pl.ds(g * LANES, LANES)
            pa16 = pa_v[crow, sl]
            v16 = val_v[crow, sl]
            b16 = lax.bitwise_and(pa16, N - 1)
            a16 = lax.shift_right_logical(pa16, 14)
            bidx3[u, gsl] = lax.shift_right_logical(b16, 2)
            q16 = lax.shift_right_logical(a16 * 52348, 25)
            waddr16 = (a16 - q16 * INTER_DIM) * OUT_DIM
            colb16 = lax.shift_left(lax.bitwise_and(b16, PACK - 1), 5)
            colp16 = colprev[u, gsl]
            colprev[u, gsl] = colb16
            for l in range(LANES):
                i = g * LANES + l
                waddr = waddr16[l]
                colb = colb16[l]
                colp = colp16[l]
                vb = jnp.full((LANES,), v16[l], jnp.float32)
                w0 = w_v[pl.ds(waddr, LANES)]
                w1 = w_v[pl.ds(waddr + LANES, LANES)]
                rows3[u, i, pl.ds(colp, LANES)] = zero16
                rows3[u, i, pl.ds(colp + LANES, LANES)] = zero16
                rows3[u, i, pl.ds(colb, LANES)] = w0 * vb
                rows3[u, i, pl.ds(colb + LANES, LANES)] = w1 * vb

    def body(j, carry):
        u = lax.rem(j, NBUF)

        @pl.when(j >= NBUF)
        def _():
            pltpu.make_async_copy(
                rows3.at[u], acc.at[bidx3.at[u]], sems.at[u]).wait()

        @pl.when(j < NBLK)
        def _():
            build(j, u)
            pltpu.make_async_copy(
                rows3.at[u], acc.at[bidx3.at[u]], sems.at[u]).start(add=True)
        return carry

    lax.fori_loop(0, NBLK + NBUF, body, 0)
    plsc.subcore_barrier()
    pltpu.sync_copy(acc.at[pl.ds(sid * ROWS_PER_TEC, ROWS_PER_TEC)],
                    out_hbm.at[cid, pl.ds(sid * ROWS_PER_TEC, ROWS_PER_TEC)])


@jax.jit
def _sc_call(pa, val, wflat, zeros):
    mesh = plsc.VectorSubcoreMesh(core_axis_name="c", subcore_axis_name="s")
    return pl.kernel(
        _sc_body,
        out_type=jax.ShapeDtypeStruct((NC, NPACK, 128), jnp.float32),
        mesh=mesh,
        scratch_types=[
            pltpu.VMEM_SHARED((NPACK, 128), jnp.float32),
            pltpu.VMEM((INTER_DIM * OUT_DIM,), jnp.float32),
            pltpu.VMEM((CROWS, 128), jnp.int32),
            pltpu.VMEM((CROWS, 128), jnp.float32),
            pltpu.VMEM((NBUF, BLK, 128), jnp.float32),
            pltpu.VMEM((NBUF, BLK), jnp.int32),
            pltpu.VMEM((NBUF, BLK), jnp.int32),
            pltpu.SemaphoreType.DMA((NBUF,)),
            pltpu.SemaphoreType.DMA((NBUF,)),
        ],
        compiler_params=pltpu.CompilerParams(needs_layout_passes=False),
    )(pa, val, wflat, zeros)


def _add_body(p_ref, o_ref):
    o_ref[...] = p_ref[0] + p_ref[1]


@jax.jit
def _tc_add(partials):
    summed = pl.pallas_call(
        _add_body,
        out_shape=jax.ShapeDtypeStruct((NPACK, 128), jnp.float32),
    )(partials)
    return summed.reshape(N, OUT_DIM)


def kernel(batch_idx, active_idx, values, f, weights):
    del f
    pa = (active_idx.astype(jnp.int32) * N
          + batch_idx.astype(jnp.int32)).reshape(NNZ // 128, 128)
    val = values.reshape(NNZ // 128, 128)
    wflat = weights.reshape(-1)
    zeros = jnp.zeros((NPACK, 128), jnp.float32)
    partials = _sc_call(pa, val, wflat, zeros)
    return _tc_add(partials)


# speedup vs baseline: 1.0231x; 1.0231x over previous
import functools

import jax
import jax.numpy as jnp
from jax import lax
from jax.experimental import pallas as pl
from jax.experimental.pallas import tpu as pltpu
from jax.experimental.pallas import tpu_sc as plsc

N = 16384
INPUT_DIM = 40960
INTER_DIM = 641
OUT_DIM = 32
NNZ = 524288

NC = 2
NS = 16
LANES = 16
NW = NC * NS
CHUNK = NNZ // NW
BLK = 64
NBLK = CHUNK // BLK
GROUPS = BLK // LANES
CROWS = CHUNK // 128
PACK = 128 // OUT_DIM
NPACK = N // PACK
ROWS_PER_TEC = NPACK // NS
NBUF = 2


def _sc_body(pa_hbm, val_hbm, w_hbm, z_hbm, out_hbm,
             acc, w_v, pa_v, val_v, rows3, bidx3, colprev, sems, zsems):
    cid = lax.axis_index("c")
    sid = lax.axis_index("s")
    wid = sid * NC + cid

    base = wid * CROWS
    pltpu.sync_copy(pa_hbm.at[pl.ds(base, CROWS)], pa_v)
    pltpu.sync_copy(val_hbm.at[pl.ds(base, CROWS)], val_v)
    pltpu.sync_copy(w_hbm, w_v)
    pltpu.sync_copy(z_hbm.at[pl.ds(sid * ROWS_PER_TEC, ROWS_PER_TEC)],
                    acc.at[pl.ds(sid * ROWS_PER_TEC, ROWS_PER_TEC)])
    for u0 in range(NBUF):
        pltpu.make_async_copy(z_hbm.at[pl.ds(0, BLK)], rows3.at[u0],
                              zsems.at[u0]).start()
    zero16i = jnp.zeros((LANES,), jnp.int32)
    zero16 = jnp.zeros((LANES,), jnp.float32)
    for u0 in range(NBUF):
        pltpu.make_async_copy(z_hbm.at[pl.ds(0, BLK)], rows3.at[u0],
                              zsems.at[u0]).wait()
        for g in range(GROUPS):
            colprev[u0, pl.ds(g * LANES, LANES)] = zero16i
    plsc.subcore_barrier()

    def build(j, u):
        crow = lax.shift_right_logical(j, 1)
        cbase = lax.shift_left(lax.bitwise_and(j, 1), 6)
        for g in range(GROUPS):
            sl = pl.ds(cbase + g * LANES, LANES)
            gsl = pl.ds(g * LANES, LANES)
            pa16 = pa_v[crow, sl]
            v16 = val_v[crow, sl]
            b16 = lax.bitwise_and(pa16, N - 1)
            a16 = lax.shift_right_logical(pa16, 14)
            bidx3[u, gsl] = lax.shift_right_logical(b16, 2)
            q16 = lax.shift_right_logical(a16 * 52348, 25)
            waddr16 = (a16 - q16 * INTER_DIM) * OUT_DIM
            colb16 = lax.shift_left(lax.bitwise_and(b16, PACK - 1), 5)
            colp16 = colprev[u, gsl]
            colprev[u, gsl] = colb16
            for l in range(LANES):
                i = g * LANES + l
                waddr = waddr16[l]
                colb = colb16[l]
                colp = colp16[l]
                vb = jnp.full((LANES,), v16[l], jnp.float32)
                w0 = w_v[pl.ds(waddr, LANES)]
                w1 = w_v[pl.ds(waddr + LANES, LANES)]
                rows3[u, i, pl.ds(colp, LANES)] = zero16
                rows3[u, i, pl.ds(colp + LANES, LANES)] = zero16
                rows3[u, i, pl.ds(colb, LANES)] = w0 * vb
                rows3[u, i, pl.ds(colb + LANES, LANES)] = w1 * vb

    def body(j, carry):
        u = lax.rem(j, NBUF)

        @pl.when(j >= NBUF)
        def _():
            pltpu.make_async_copy(
                rows3.at[u], acc.at[bidx3.at[u]], sems.at[u]).wait()

        @pl.when(j < NBLK)
        def _():
            build(j, u)
            pltpu.make_async_copy(
                rows3.at[u], acc.at[bidx3.at[u]], sems.at[u]).start(add=True)
        return carry

    lax.fori_loop(0, NBLK + NBUF, body, 0)
    plsc.subcore_barrier()
    pltpu.sync_copy(acc.at[pl.ds(sid * ROWS_PER_TEC, ROWS_PER_TEC)],
                    out_hbm.at[cid, pl.ds(sid * ROWS_PER_TEC, ROWS_PER_TEC)])


@jax.jit
def _sc_call(pa, val, wflat, zeros):
    mesh = plsc.VectorSubcoreMesh(core_axis_name="c", subcore_axis_name="s")
    return pl.kernel(
        _sc_body,
        out_type=jax.ShapeDtypeStruct((NC, NPACK, 128), jnp.float32),
        mesh=mesh,
        scratch_types=[
            pltpu.VMEM_SHARED((NPACK, 128), jnp.float32),
            pltpu.VMEM((INTER_DIM * OUT_DIM,), jnp.float32),
            pltpu.VMEM((CROWS, 128), jnp.int32),
            pltpu.VMEM((CROWS, 128), jnp.float32),
            pltpu.VMEM((NBUF, BLK, 128), jnp.float32),
            pltpu.VMEM((NBUF, BLK), jnp.int32),
            pltpu.VMEM((NBUF, BLK), jnp.int32),
            pltpu.SemaphoreType.DMA((NBUF,)),
            pltpu.SemaphoreType.DMA((NBUF,)),
        ],
        compiler_params=pltpu.CompilerParams(needs_layout_passes=False),
    )(pa, val, wflat, zeros)


def _add_body(p_ref, o_ref):
    o_ref[...] = p_ref[0] + p_ref[1]


@jax.jit
def _tc_add(partials):
    summed = pl.pallas_call(
        _add_body,
        out_shape=jax.ShapeDtypeStruct((NPACK, 128), jnp.float32),
    )(partials)
    return summed.reshape(N, OUT_DIM)


def kernel(batch_idx, active_idx, values, f, weights):
    del f
    pa = (active_idx.astype(jnp.int32) * N
          + batch_idx.astype(jnp.int32)).reshape(NNZ // 128, 128)
    val = values.reshape(NNZ // 128, 128)
    wflat = weights.reshape(-1)
    zeros = jnp.zeros((NPACK, 128), jnp.float32)
    partials = _sc_call(pa, val, wflat, zeros)
    return _tc_add(partials)
